# baseline (speedup 1.0000x reference)
import functools
import math

import jax
import jax.numpy as jnp
from jax import lax
from jax.experimental import pallas as pl
from jax.experimental.pallas import tpu as pltpu
from jax.experimental.pallas import tpu_sc as plsc

_VOCAB = 151936
_D = 1024
_B = 2
_S = 4096
_H = 16
_HEAD = _D // _H
_THETA = 1000000.0

_N = _B * _S
_NC = 2
_NS = 16
_NW = _NC * _NS
_PER_W = _N // _NW
_CHUNK = 32
_NCH = _PER_W // _CHUNK
_NBUF = 3


def _gather_body(ids_hbm, table_hbm, out_hbm, idx_v, rows_v, gsem, wsem):
    wid = lax.axis_index("s") * _NC + lax.axis_index("c")
    wpb = _S // _PER_W
    b = wid // wpb
    s0 = (wid % wpb) * _PER_W
    pltpu.sync_copy(ids_hbm.at[b, pl.ds(s0, _PER_W)], idx_v)

    def gather(c):
        return pltpu.async_copy(table_hbm.at[idx_v.at[pl.ds(c * _CHUNK, _CHUNK)]],
                                rows_v.at[c % _NBUF], gsem)

    def write(c):
        return pltpu.async_copy(rows_v.at[c % _NBUF],
                                out_hbm.at[b, pl.ds(s0 + c * _CHUNK, _CHUNK)],
                                wsem)

    gathers = [gather(0), gather(1)]
    writes = []
    for c in range(_NCH):
        gathers[c].wait()
        if c + 2 < _NCH:
            if c >= 1:
                writes[c - 1].wait()
            gathers.append(gather(c + 2))
        writes.append(write(c))
    for c in range(max(0, _NCH - 3), _NCH):
        writes[c].wait()


def _rope_body(inv_ref, cos_ref, sin_ref, pos_ref):
    pos = lax.broadcasted_iota(jnp.int32, (1, _S, _HEAD), 1)
    ang = pos.astype(jnp.float32) * inv_ref[...]
    cos_ref[...] = jnp.cos(ang)
    sin_ref[...] = jnp.sin(ang)
    pos_ref[...] = lax.broadcasted_iota(jnp.int32, (1, _S), 1)


def kernel(input_ids, labels, W):
    @functools.partial(
        pl.kernel,
        out_type=jax.ShapeDtypeStruct((_B, _S, _D), jnp.float32),
        mesh=plsc.VectorSubcoreMesh(core_axis_name="c", subcore_axis_name="s"),
        scratch_types=[
            pltpu.VMEM((_PER_W,), jnp.int32),
            pltpu.VMEM((_NBUF, _CHUNK, _D), jnp.float32),
            pltpu.SemaphoreType.DMA,
            pltpu.SemaphoreType.DMA,
        ],
    )
    def gather_sc(ids_hbm, table_hbm, out_hbm, idx_v, rows_v, gsem, wsem):
        _gather_body(ids_hbm, table_hbm, out_hbm, idx_v, rows_v, gsem, wsem)

    hidden_states = gather_sc(input_ids, W)

    half = jnp.arange(0, _HEAD, 2, dtype=jnp.float32) / _HEAD
    inv_freq = 1.0 / (_THETA ** half)
    inv_full = jnp.concatenate([inv_freq, inv_freq])[None, None, :]

    cos, sin, position_ids = pl.pallas_call(
        _rope_body,
        out_shape=[
            jax.ShapeDtypeStruct((1, _S, _HEAD), jnp.float32),
            jax.ShapeDtypeStruct((1, _S, _HEAD), jnp.float32),
            jax.ShapeDtypeStruct((1, _S), jnp.int32),
        ],
    )(inv_full)

    requires_grad_idx = jnp.array([3], dtype=jnp.int32)
    cache_position = position_ids.reshape(_S)
    return (requires_grad_idx, cos, sin, hidden_states, position_ids,
            cache_position, labels)
